# baseline (speedup 1.0000x reference)
import functools

import jax
import jax.numpy as jnp
from jax import lax
from jax.experimental import pallas as pl
from jax.experimental.pallas import tpu as pltpu
from jax.experimental.pallas import tpu_sc as plsc

_B = 16384
_D = 64
_NC = 2
_NS = 16
_NW = _NC * _NS
_BPW = _B // _NW


def _sc_gather(U, gmf_tab, mlp_tab, bias_tab):
    mesh = plsc.VectorSubcoreMesh(core_axis_name="c", subcore_axis_name="s")

    @functools.partial(
        pl.kernel,
        mesh=mesh,
        compiler_params=pltpu.CompilerParams(use_tc_tiling_on_sc=False),
        out_type=(
            jax.ShapeDtypeStruct((_B, _D), jnp.float32),
            jax.ShapeDtypeStruct((_B, _D), jnp.float32),
            jax.ShapeDtypeStruct((_B,), jnp.float32),
        ),
        scratch_types=[
            pltpu.VMEM((_BPW,), jnp.int32),
            pltpu.VMEM((_BPW, _D), jnp.float32),
            pltpu.VMEM((_BPW, _D), jnp.float32),
            pltpu.VMEM((_BPW,), jnp.float32),
            pltpu.SemaphoreType.DMA,
            pltpu.SemaphoreType.DMA,
            pltpu.SemaphoreType.DMA,
        ],
    )
    def k(u_hbm, g_hbm, m_hbm, b_hbm, out_g, out_m, out_b,
          idx_v, rows_g, rows_m, rows_b, sem_g, sem_m, sem_b):
        wid = lax.axis_index("s") * _NC + lax.axis_index("c")
        base = wid * _BPW
        pltpu.sync_copy(u_hbm.at[pl.ds(base, _BPW)], idx_v)
        cg = pltpu.async_copy(g_hbm.at[idx_v], rows_g, sem_g)
        cm = pltpu.async_copy(m_hbm.at[idx_v], rows_m, sem_m)
        cb = pltpu.async_copy(b_hbm.at[idx_v], rows_b, sem_b)
        cg.wait()
        pltpu.sync_copy(rows_g, out_g.at[pl.ds(base, _BPW)])
        cm.wait()
        pltpu.sync_copy(rows_m, out_m.at[pl.ds(base, _BPW)])
        cb.wait()
        pltpu.sync_copy(rows_b, out_b.at[pl.ds(base, _BPW)])

    return k(U, gmf_tab, mlp_tab, bias_tab.reshape(-1))


def _dense_body(e_ref, ug_ref, um_ref, ub_ref,
                wg_ref, bg_ref, wm_ref, bm_ref,
                w0a_ref, w0b_ref, b0_ref, w1_ref, b1_ref, w2_ref, b2_ref,
                wfg_ref, wfh_ref, wib_ref, c_ref, out_ref):
    e = e_ref[:]
    item_g = jnp.dot(e, wg_ref[:], preferred_element_type=jnp.float32) + bg_ref[:]
    gmf = ug_ref[:] * item_g
    item_m = jnp.dot(e, wm_ref[:], preferred_element_type=jnp.float32) + bm_ref[:]
    h = (jnp.dot(um_ref[:], w0a_ref[:], preferred_element_type=jnp.float32)
         + jnp.dot(item_m, w0b_ref[:], preferred_element_type=jnp.float32)
         + b0_ref[:])
    h = jnp.maximum(h, 0.0)
    h = jnp.maximum(jnp.dot(h, w1_ref[:], preferred_element_type=jnp.float32) + b1_ref[:], 0.0)
    h = jnp.maximum(jnp.dot(h, w2_ref[:], preferred_element_type=jnp.float32) + b2_ref[:], 0.0)
    pred = (jnp.sum(gmf * wfg_ref[:][None, :], axis=1)
            + jnp.sum(h * wfh_ref[:][None, :], axis=1)
            + jnp.sum(e * wib_ref[:][None, :], axis=1))
    pred = pred + ub_ref[:] + c_ref[0]
    out_ref[:] = jax.nn.sigmoid(pred)


def _tc_dense(E, ug, um, ub, Wg, bg, Wm, bm, W0a, W0b, b0, W1t, b1, W2t, b2,
              wfg, wfh, wib, c):
    grid = 8
    r = _B // grid

    def row2(d):
        return pl.BlockSpec((r, d), lambda i: (i, 0))

    row1 = pl.BlockSpec((r,), lambda i: (i,))

    def full2(a):
        return pl.BlockSpec(a.shape, lambda i: (0, 0))

    def full1(a):
        return pl.BlockSpec(a.shape, lambda i: (0,))

    return pl.pallas_call(
        _dense_body,
        grid=(grid,),
        in_specs=[row2(_D), row2(_D), row2(_D), row1,
                  full2(Wg), full1(bg), full2(Wm), full1(bm),
                  full2(W0a), full2(W0b), full1(b0),
                  full2(W1t), full1(b1), full2(W2t), full1(b2),
                  full1(wfg), full1(wfh), full1(wib), full1(c)],
        out_specs=row1,
        out_shape=jax.ShapeDtypeStruct((_B,), jnp.float32),
    )(E, ug, um, ub, Wg, bg, Wm, bm, W0a, W0b, b0, W1t, b1, W2t, b2,
      wfg, wfh, wib, c)


def kernel(U, E, user_emb_gmf, user_bias_tab, W_item_gmf, b_item_gmf,
           W_item_bias, b_item_bias, user_emb_mlp, W_item_mlp, b_item_mlp,
           W_mlp0, b_mlp0, W_mlp1, b_mlp1, W_mlp2, b_mlp2, W_final, b_final):
    u32 = U.astype(jnp.int32)
    ug, um, ub = _sc_gather(u32, user_emb_gmf, user_emb_mlp, user_bias_tab)
    Wg = W_item_gmf.T
    Wm = W_item_mlp.T
    W0a = W_mlp0[:, :_D].T
    W0b = W_mlp0[:, _D:].T
    W1t = W_mlp1.T
    W2t = W_mlp2.T
    wfg = W_final[0, :_D]
    wfh = W_final[0, _D:]
    wib = W_item_bias[0]
    c = b_final + b_item_bias
    return _tc_dense(E, ug, um, ub, Wg, b_item_gmf, Wm, b_item_mlp,
                     W0a, W0b, b_mlp0, W1t, b_mlp1, W2t, b_mlp2,
                     wfg, wfh, wib, c)
